# baseline (speedup 1.0000x reference)
import functools

import jax
import jax.numpy as jnp
from jax.experimental import pallas as pl
from jax.experimental.pallas import tpu as pltpu
from jax.experimental.pallas import tpu_sc as plsc

E = 8
D = 768
H = 2048
T = 2048
B = 256
NB = T // B + E
PAD_T = NB * B
CH = 128
NW = 32
RPW = T // NW


def _route_body(eid_ref, dest_ref, be_ref):
    eid = eid_ref[...]
    lane = jax.lax.broadcasted_iota(jnp.int32, (T, E), 1)
    onehot = (lane == eid).astype(jnp.float32)

    counts = jnp.sum(onehot, axis=0, keepdims=True)
    padded = jnp.floor((counts + (B - 1)) * (1.0 / B)) * B
    upper = (jax.lax.broadcasted_iota(jnp.int32, (E, E), 0)
             < jax.lax.broadcasted_iota(jnp.int32, (E, E), 1))
    excl = jax.lax.dot_general(
        padded, upper.astype(jnp.float32), (((1,), (0,)), ((), ())))

    bstart = excl * (1.0 / B)
    bi = jax.lax.broadcasted_iota(jnp.int32, (NB, E), 0).astype(jnp.float32)
    be = jnp.sum((bi >= bstart).astype(jnp.float32), axis=1, keepdims=True) - 1.0
    be_ref[...] = be.astype(jnp.int32)

    ltri = (jax.lax.broadcasted_iota(jnp.int32, (CH, CH), 0)
            > jax.lax.broadcasted_iota(jnp.int32, (CH, CH), 1)).astype(jnp.float32)
    carry = jnp.zeros((1, E), jnp.float32)
    for c in range(T // CH):
        oh = onehot[c * CH:(c + 1) * CH]
        prefix = jax.lax.dot_general(ltri, oh, (((1,), (0,)), ((), ())))
        d = jnp.sum(oh * (prefix + carry + excl), axis=1, keepdims=True)
        dest_ref[c * CH:(c + 1) * CH, :] = d.astype(jnp.int32)
        carry = carry + jnp.sum(oh, axis=0, keepdims=True)


_route = pl.pallas_call(
    _route_body,
    out_shape=(jax.ShapeDtypeStruct((T, 1), jnp.int32),
               jax.ShapeDtypeStruct((NB, 1), jnp.int32)),
)


def _ffn_body(be_ref, xs_ref, w1_ref, w3_ref, w2_ref, out_ref):
    del be_ref
    xb = xs_ref[...].astype(jnp.bfloat16)
    w1 = w1_ref[0]
    w3 = w3_ref[0]
    w2 = w2_ref[0]
    a = jax.lax.dot_general(xb, w1, (((1,), (1,)), ((), ())),
                            preferred_element_type=jnp.float32)
    g = jax.lax.dot_general(xb, w3, (((1,), (1,)), ((), ())),
                            preferred_element_type=jnp.float32)
    h = (a * jax.nn.sigmoid(a) * g).astype(jnp.bfloat16)
    out_ref[...] = jax.lax.dot_general(h, w2, (((1,), (1,)), ((), ())),
                                       preferred_element_type=jnp.float32)


_ffn = pl.pallas_call(
    _ffn_body,
    grid_spec=pltpu.PrefetchScalarGridSpec(
        num_scalar_prefetch=1,
        grid=(NB,),
        in_specs=[
            pl.BlockSpec((B, D), lambda i, be: (i, 0)),
            pl.BlockSpec((1, H, D), lambda i, be: (be[i], 0, 0)),
            pl.BlockSpec((1, H, D), lambda i, be: (be[i], 0, 0)),
            pl.BlockSpec((1, D, H), lambda i, be: (be[i], 0, 0)),
        ],
        out_specs=pl.BlockSpec((B, D), lambda i, be: (i, 0)),
    ),
    out_shape=jax.ShapeDtypeStruct((PAD_T, D), jnp.float32),
)


@functools.lru_cache(maxsize=1)
def _sc_kernels():
    mesh = plsc.VectorSubcoreMesh(core_axis_name="c", subcore_axis_name="s")
    sc_scratch = [
        pltpu.VMEM((RPW,), jnp.int32),
        pltpu.VMEM((RPW, D), jnp.float32),
        pltpu.SemaphoreType.DMA,
    ]

    @functools.partial(
        pl.kernel,
        mesh=mesh,
        out_type=jax.ShapeDtypeStruct((PAD_T, D), jnp.float32),
        scratch_types=sc_scratch,
    )
    def dispatch(x_hbm, dest_hbm, out_hbm, idx_v, rows_v, sem):
        wid = jax.lax.axis_index("s") * 2 + jax.lax.axis_index("c")
        base = wid * RPW
        pltpu.sync_copy(dest_hbm.at[pl.ds(base, RPW)], idx_v)
        pltpu.sync_copy(x_hbm.at[pl.ds(base, RPW)], rows_v)
        pltpu.async_copy(rows_v, out_hbm.at[idx_v], sem).wait()

    @functools.partial(
        pl.kernel,
        mesh=mesh,
        out_type=jax.ShapeDtypeStruct((T, D), jnp.float32),
        scratch_types=sc_scratch,
    )
    def combine(ys_hbm, dest_hbm, out_hbm, idx_v, rows_v, sem):
        wid = jax.lax.axis_index("s") * 2 + jax.lax.axis_index("c")
        base = wid * RPW
        pltpu.sync_copy(dest_hbm.at[pl.ds(base, RPW)], idx_v)
        pltpu.async_copy(ys_hbm.at[idx_v], rows_v, sem).wait()
        pltpu.sync_copy(rows_v, out_hbm.at[pl.ds(base, RPW)])

    return dispatch, combine


def kernel(x, Wg, w1, w2, w3):
    orig_shape = x.shape
    xf = x.reshape(-1, x.shape[-1])
    scores = xf @ Wg.T
    _, expert_indices = jax.lax.top_k(scores, 1)
    dest2, be2 = _route(expert_indices)
    dest = dest2.reshape(T)
    be = be2.reshape(NB)
    dispatch, combine = _sc_kernels()
    xs = dispatch(xf, dest)
    ys = _ffn(be, xs, w1.astype(jnp.bfloat16), w3.astype(jnp.bfloat16),
              w2.astype(jnp.bfloat16))
    y = combine(ys, dest)
    return y.reshape(orig_shape)


# speedup vs baseline: 1.2060x; 1.2060x over previous
import functools

import jax
import jax.numpy as jnp
from jax.experimental import pallas as pl
from jax.experimental.pallas import tpu as pltpu
from jax.experimental.pallas import tpu_sc as plsc

E = 8
D = 768
H = 2048
T = 2048
B = 256
NB = T // B + E
PAD_T = NB * B
CH = 128
NW = 32
RPW = T // NW


def _route_body(eid_ref, dest_ref, be_ref):
    eid = eid_ref[...]
    lane = jax.lax.broadcasted_iota(jnp.int32, (T, E), 1)
    onehot = (lane == eid).astype(jnp.float32)

    counts = jnp.sum(onehot, axis=0, keepdims=True)
    padded = jnp.floor((counts + (B - 1)) * (1.0 / B)) * B
    upper = (jax.lax.broadcasted_iota(jnp.int32, (E, E), 0)
             < jax.lax.broadcasted_iota(jnp.int32, (E, E), 1))
    excl = jax.lax.dot_general(
        padded, upper.astype(jnp.float32), (((1,), (0,)), ((), ())))

    bstart = excl * (1.0 / B)
    bi = jax.lax.broadcasted_iota(jnp.int32, (NB, E), 0).astype(jnp.float32)
    be = jnp.sum((bi >= bstart).astype(jnp.float32), axis=1, keepdims=True) - 1.0
    be_ref[...] = be.astype(jnp.int32)

    ltri = (jax.lax.broadcasted_iota(jnp.int32, (CH, CH), 0)
            > jax.lax.broadcasted_iota(jnp.int32, (CH, CH), 1)).astype(jnp.float32)
    carry = jnp.zeros((1, E), jnp.float32)
    for c in range(T // CH):
        oh = onehot[c * CH:(c + 1) * CH]
        prefix = jax.lax.dot_general(ltri, oh, (((1,), (0,)), ((), ())))
        d = jnp.sum(oh * (prefix + carry + excl), axis=1, keepdims=True)
        dest_ref[c * CH:(c + 1) * CH, :] = d.astype(jnp.int32)
        carry = carry + jnp.sum(oh, axis=0, keepdims=True)


_route = pl.pallas_call(
    _route_body,
    out_shape=(jax.ShapeDtypeStruct((T, 1), jnp.int32),
               jax.ShapeDtypeStruct((NB, 1), jnp.int32)),
)


def _ffn_body(be_ref, xs_ref, w1_ref, w3_ref, w2_ref, out_ref):
    del be_ref
    xb = xs_ref[...].astype(jnp.bfloat16)
    w1 = w1_ref[0].astype(jnp.bfloat16)
    w3 = w3_ref[0].astype(jnp.bfloat16)
    w2 = w2_ref[0].astype(jnp.bfloat16)
    a = jax.lax.dot_general(xb, w1, (((1,), (1,)), ((), ())),
                            preferred_element_type=jnp.float32)
    g = jax.lax.dot_general(xb, w3, (((1,), (1,)), ((), ())),
                            preferred_element_type=jnp.float32)
    h = (a * jax.nn.sigmoid(a) * g).astype(jnp.bfloat16)
    out_ref[...] = jax.lax.dot_general(h, w2, (((1,), (1,)), ((), ())),
                                       preferred_element_type=jnp.float32)


_ffn = pl.pallas_call(
    _ffn_body,
    grid_spec=pltpu.PrefetchScalarGridSpec(
        num_scalar_prefetch=1,
        grid=(NB,),
        in_specs=[
            pl.BlockSpec((B, D), lambda i, be: (i, 0)),
            pl.BlockSpec((1, H, D), lambda i, be: (be[i], 0, 0)),
            pl.BlockSpec((1, H, D), lambda i, be: (be[i], 0, 0)),
            pl.BlockSpec((1, D, H), lambda i, be: (be[i], 0, 0)),
        ],
        out_specs=pl.BlockSpec((B, D), lambda i, be: (i, 0)),
    ),
    out_shape=jax.ShapeDtypeStruct((PAD_T, D), jnp.float32),
)


@functools.lru_cache(maxsize=1)
def _sc_kernels():
    mesh = plsc.VectorSubcoreMesh(core_axis_name="c", subcore_axis_name="s")
    sc_scratch = [
        pltpu.VMEM((RPW,), jnp.int32),
        pltpu.VMEM((RPW, D), jnp.float32),
        pltpu.SemaphoreType.DMA,
    ]

    @functools.partial(
        pl.kernel,
        mesh=mesh,
        out_type=jax.ShapeDtypeStruct((PAD_T, D), jnp.float32),
        scratch_types=sc_scratch,
    )
    def dispatch(x_hbm, dest_hbm, out_hbm, idx_v, rows_v, sem):
        wid = jax.lax.axis_index("s") * 2 + jax.lax.axis_index("c")
        base = wid * RPW
        pltpu.sync_copy(dest_hbm.at[pl.ds(base, RPW)], idx_v)
        pltpu.sync_copy(x_hbm.at[pl.ds(base, RPW)], rows_v)
        pltpu.async_copy(rows_v, out_hbm.at[idx_v], sem).wait()

    @functools.partial(
        pl.kernel,
        mesh=mesh,
        out_type=jax.ShapeDtypeStruct((T, D), jnp.float32),
        scratch_types=sc_scratch,
    )
    def combine(ys_hbm, dest_hbm, out_hbm, idx_v, rows_v, sem):
        wid = jax.lax.axis_index("s") * 2 + jax.lax.axis_index("c")
        base = wid * RPW
        pltpu.sync_copy(dest_hbm.at[pl.ds(base, RPW)], idx_v)
        pltpu.async_copy(ys_hbm.at[idx_v], rows_v, sem).wait()
        pltpu.sync_copy(rows_v, out_hbm.at[pl.ds(base, RPW)])

    return dispatch, combine


def kernel(x, Wg, w1, w2, w3):
    orig_shape = x.shape
    xf = x.reshape(-1, x.shape[-1])
    scores = xf @ Wg.T
    _, expert_indices = jax.lax.top_k(scores, 1)
    dest2, be2 = _route(expert_indices)
    dest = dest2.reshape(T)
    be = be2.reshape(NB)
    dispatch, combine = _sc_kernels()
    xs = dispatch(xf, dest)
    ys = _ffn(be, xs, w1, w3, w2)
    y = combine(ys, dest)
    return y.reshape(orig_shape)
